# Initial kernel scaffold; baseline (speedup 1.0000x reference)
import jax
import jax.numpy as jnp
from jax.experimental import pallas as pl


def kernel(x, edge_index, query_embedding, W1, b1, W2, b2, Wfc, bfc):
    raise NotImplementedError("write your pallas kernel here")


# speedup vs baseline: 12.7851x; 12.7851x over previous
import functools

import jax
import jax.numpy as jnp
from jax import lax
from jax.experimental import pallas as pl
from jax.experimental.pallas import tpu as pltpu
from jax.experimental.pallas import tpu_sc as plsc

N = 10000
E = 160000
H = 256
HH = H // 2
NC, NS = 2, 16
EB = 125
NB = (E // NS) // EB
DEG_NB = (E // (NC * NS)) // EB
RPT = 624
REM = N - NS * RPT

RB = 1000
GRID = N // RB

_mesh = plsc.VectorSubcoreMesh(core_axis_name="c", subcore_axis_name="s")


def _deg_body(dst_hbm, ones_hbm, out_hbm, didx, ones_v, deg_sh):
    c = lax.axis_index("c")
    s = lax.axis_index("s")
    pltpu.sync_copy(ones_hbm, ones_v)
    @pl.loop(0, 4)
    def _(i):
        pltpu.sync_copy(ones_v.at[pl.ds(0, 156)],
                        deg_sh.at[pl.ds(s * RPT + i * 156, 156)])

    @pl.when(s == NS - 1)
    def _():
        pltpu.sync_copy(ones_v.at[pl.ds(0, REM)],
                        deg_sh.at[pl.ds(NS * RPT, REM)])

    pltpu.sync_copy(dst_hbm.at[c, s], didx)
    plsc.subcore_barrier()

    @pl.loop(0, DEG_NB)
    def _(b):
        pltpu.sync_copy(ones_v.at[pl.ds(0, EB)], deg_sh.at[didx.at[b]], add=True)

    plsc.subcore_barrier()
    pltpu.sync_copy(deg_sh.at[pl.ds(s * RPT, RPT)],
                    out_hbm.at[c, pl.ds(s * RPT, RPT)])

    @pl.when(s == NS - 1)
    def _():
        pltpu.sync_copy(deg_sh.at[pl.ds(NS * RPT, REM)],
                        out_hbm.at[c, pl.ds(NS * RPT, REM)])


_deg_call = pl.kernel(
    _deg_body,
    out_type=jax.ShapeDtypeStruct((NC, N, HH), jnp.float32),
    mesh=_mesh,
    scratch_types=[
        pltpu.VMEM((DEG_NB, EB), jnp.int32),
        pltpu.VMEM((156, HH), jnp.float32),
        pltpu.VMEM_SHARED((N, HH), jnp.float32),
    ],
)


def _scatter_body(xws_hbm, src_hbm, dst_hbm, out_hbm, sidx, didx, rows, accum_sh, sem):
    c = lax.axis_index("c")
    s = lax.axis_index("s")
    pltpu.sync_copy(xws_hbm.at[pl.ds(c * N + s * RPT, RPT)],
                    accum_sh.at[pl.ds(s * RPT, RPT)])

    @pl.when(s == NS - 1)
    def _():
        pltpu.sync_copy(xws_hbm.at[pl.ds(c * N + NS * RPT, REM)],
                        accum_sh.at[pl.ds(NS * RPT, REM)])

    pltpu.sync_copy(src_hbm.at[c, s], sidx)
    pltpu.sync_copy(dst_hbm.at[s], didx)
    plsc.subcore_barrier()

    @pl.loop(0, NB)
    def _(b):
        pltpu.async_copy(xws_hbm.at[sidx.at[b]], rows, sem).wait()
        pltpu.sync_copy(rows, accum_sh.at[didx.at[b]], add=True)

    plsc.subcore_barrier()
    pltpu.sync_copy(accum_sh.at[pl.ds(s * RPT, RPT)],
                    out_hbm.at[pl.ds(c * N + s * RPT, RPT)])

    @pl.when(s == NS - 1)
    def _():
        pltpu.sync_copy(accum_sh.at[pl.ds(NS * RPT, REM)],
                        out_hbm.at[pl.ds(c * N + NS * RPT, REM)])


_scatter_call = pl.kernel(
    _scatter_body,
    out_type=jax.ShapeDtypeStruct((NC * N, HH), jnp.float32),
    mesh=_mesh,
    scratch_types=[
        pltpu.VMEM((NB, EB), jnp.int32),
        pltpu.VMEM((NB, EB), jnp.int32),
        pltpu.VMEM((EB, HH), jnp.float32),
        pltpu.VMEM_SHARED((N, HH), jnp.float32),
        pltpu.SemaphoreType.DMA,
    ],
)


def _dinv(degp_ref):
    return lax.rsqrt(degp_ref[0, :, 0:1] + degp_ref[1, :, 0:1] - 1.0)


def _mm1_body(x_ref, w_ref, degp_ref, out_ref):
    xw = jnp.dot(x_ref[...], w_ref[...], preferred_element_type=jnp.float32)
    d = _dinv(degp_ref)
    out_ref[0] = xw[:, :HH] * d
    out_ref[1] = xw[:, HH:] * d


def _mid_body(a_ref, degp_ref, b1_ref, w_ref, out_ref):
    d = _dinv(degp_ref)
    h0 = jnp.maximum(a_ref[0] * d + b1_ref[0, :HH], 0.0)
    h1 = jnp.maximum(a_ref[1] * d + b1_ref[0, HH:], 0.0)
    xw = (jnp.dot(h0, w_ref[:HH, :], preferred_element_type=jnp.float32)
          + jnp.dot(h1, w_ref[HH:, :], preferred_element_type=jnp.float32))
    out_ref[0] = xw[:, :HH] * d
    out_ref[1] = xw[:, HH:] * d


def _fin_body(a_ref, degp_ref, b2_ref, wfc_ref, q_ref, bfc_ref, out_ref):
    d = _dinv(degp_ref)
    h0 = jnp.maximum(a_ref[0] * d + b2_ref[0, :HH], 0.0)
    h1 = jnp.maximum(a_ref[1] * d + b2_ref[0, HH:], 0.0)
    sc = (jnp.dot(h0, wfc_ref[:HH, :], preferred_element_type=jnp.float32)
          + jnp.dot(h1, wfc_ref[HH:H, :], preferred_element_type=jnp.float32))
    const = jnp.sum(q_ref[0, :] * wfc_ref[H:, 0]) + bfc_ref[0, 0]
    out_ref[...] = sc + const


def _row_specs():
    degp = pl.BlockSpec((NC, RB, HH), lambda i: (0, i, 0))
    half = pl.BlockSpec((NC, RB, HH), lambda i: (0, i, 0))
    return degp, half


def _mm1(x, W1, degp):
    dspec, half = _row_specs()
    return pl.pallas_call(
        _mm1_body,
        grid=(GRID,),
        in_specs=[
            pl.BlockSpec((RB, H), lambda i: (i, 0)),
            pl.BlockSpec((H, H), lambda i: (0, 0)),
            dspec,
        ],
        out_specs=half,
        out_shape=jax.ShapeDtypeStruct((NC, N, HH), jnp.float32),
    )(x, W1, degp)


def _mid(a, degp, b1r, W2):
    dspec, half = _row_specs()
    return pl.pallas_call(
        _mid_body,
        grid=(GRID,),
        in_specs=[
            half,
            dspec,
            pl.BlockSpec((1, H), lambda i: (0, 0)),
            pl.BlockSpec((H, H), lambda i: (0, 0)),
        ],
        out_specs=half,
        out_shape=jax.ShapeDtypeStruct((NC, N, HH), jnp.float32),
    )(a, degp, b1r, W2)


def _fin(a, degp, b2r, Wfc, qr, bfcr):
    dspec, half = _row_specs()
    return pl.pallas_call(
        _fin_body,
        grid=(GRID,),
        in_specs=[
            half,
            dspec,
            pl.BlockSpec((1, H), lambda i: (0, 0)),
            pl.BlockSpec((2 * H, 1), lambda i: (0, 0)),
            pl.BlockSpec((1, H), lambda i: (0, 0)),
            pl.BlockSpec((1, 1), lambda i: (0, 0)),
        ],
        out_specs=pl.BlockSpec((RB, 1), lambda i: (i, 0)),
        out_shape=jax.ShapeDtypeStruct((N, 1), jnp.float32),
    )(a, degp, b2r, Wfc, qr, bfcr)


@jax.jit
def kernel(x, edge_index, query_embedding, W1, b1, W2, b2, Wfc, bfc):
    src = edge_index[0].astype(jnp.int32)
    dst = edge_index[1].astype(jnp.int32)
    src4 = jnp.stack([src, src + N]).reshape(NC, NS, NB, EB)
    dst3 = dst.reshape(NS, NB, EB)
    dstd = dst.reshape(NC, NS, DEG_NB, EB)
    ones_hbm = jnp.ones((156, HH), jnp.float32)

    degp = _deg_call(dstd, ones_hbm)
    xws1 = _mm1(x, W1, degp)
    a1 = _scatter_call(xws1.reshape(NC * N, HH), src4, dst3)
    xws2 = _mid(a1.reshape(NC, N, HH), degp, b1.reshape(1, H), W2)
    a2 = _scatter_call(xws2.reshape(NC * N, HH), src4, dst3)
    out = _fin(a2.reshape(NC, N, HH), degp, b2.reshape(1, H), Wfc,
               query_embedding.reshape(1, H), bfc.reshape(1, 1))
    return out.reshape(N)


# speedup vs baseline: 17.8335x; 1.3949x over previous
import functools

import jax
import jax.numpy as jnp
from jax import lax
from jax.experimental import pallas as pl
from jax.experimental.pallas import tpu as pltpu
from jax.experimental.pallas import tpu_sc as plsc

N = 10000
E = 160000
H = 256
HH = H // 2
NC, NS = 2, 16
EB = 125
DEG_NB = (E // (NC * NS)) // EB
NB = (E // NS) // EB
NPH = 2
PB = NB // NPH
RPT = 624
REM = N - NS * RPT

RB = 1000
GRID = N // RB

_mesh = plsc.VectorSubcoreMesh(core_axis_name="c", subcore_axis_name="s")


def _deg_body(dst_hbm, ones_hbm, out_hbm, didx, ones_v, deg_sh):
    c = lax.axis_index("c")
    s = lax.axis_index("s")
    pltpu.sync_copy(ones_hbm, ones_v)
    @pl.loop(0, 4)
    def _(i):
        pltpu.sync_copy(ones_v.at[pl.ds(0, 156)],
                        deg_sh.at[pl.ds(s * RPT + i * 156, 156)])

    @pl.when(s == NS - 1)
    def _():
        pltpu.sync_copy(ones_v.at[pl.ds(0, REM)],
                        deg_sh.at[pl.ds(NS * RPT, REM)])

    pltpu.sync_copy(dst_hbm.at[c, s], didx)
    plsc.subcore_barrier()

    @pl.loop(0, DEG_NB)
    def _(b):
        pltpu.sync_copy(ones_v.at[pl.ds(0, EB)], deg_sh.at[didx.at[b]],
                        add=True)

    plsc.subcore_barrier()
    pltpu.sync_copy(deg_sh.at[pl.ds(s * RPT, RPT)],
                    out_hbm.at[c, pl.ds(s * RPT, RPT)])

    @pl.when(s == NS - 1)
    def _():
        pltpu.sync_copy(deg_sh.at[pl.ds(NS * RPT, REM)],
                        out_hbm.at[c, pl.ds(NS * RPT, REM)])


_deg_call = pl.kernel(
    _deg_body,
    out_type=jax.ShapeDtypeStruct((NC, N, HH), jnp.float32),
    mesh=_mesh,
    scratch_types=[
        pltpu.VMEM((DEG_NB, EB), jnp.int32),
        pltpu.VMEM((156, HH), jnp.float32),
        pltpu.VMEM_SHARED((N, HH), jnp.float32),
    ],
)


def _scatter_body(xws_hbm, src_hbm, dst_hbm, out_hbm, sidx, didx, rows,
                  accum_sh, sem0, sem1):
    rows0, rows1 = rows.at[0], rows.at[1]
    c = lax.axis_index("c")
    s = lax.axis_index("s")
    pltpu.sync_copy(xws_hbm.at[pl.ds(c * N + s * RPT, RPT)],
                    accum_sh.at[pl.ds(s * RPT, RPT)])

    @pl.when(s == NS - 1)
    def _():
        pltpu.sync_copy(xws_hbm.at[pl.ds(c * N + NS * RPT, REM)],
                        accum_sh.at[pl.ds(NS * RPT, REM)])

    plsc.subcore_barrier()

    for p in range(NPH):
        pltpu.sync_copy(src_hbm.at[c, s, pl.ds(p * PB, PB)], sidx)
        pltpu.sync_copy(dst_hbm.at[s, pl.ds(p * PB, PB)], didx)
        pltpu.async_copy(xws_hbm.at[sidx.at[0]], rows0, sem0)
        pltpu.async_copy(xws_hbm.at[sidx.at[1]], rows1, sem1)

        @pl.loop(0, PB - 2, step=2)
        def _(g):
            pltpu.make_async_copy(xws_hbm.at[sidx.at[g]], rows0, sem0).wait()
            pltpu.sync_copy(rows0, accum_sh.at[didx.at[g]], add=True)
            pltpu.async_copy(xws_hbm.at[sidx.at[g + 2]], rows0, sem0)
            pltpu.make_async_copy(xws_hbm.at[sidx.at[g + 1]], rows1, sem1).wait()
            pltpu.sync_copy(rows1, accum_sh.at[didx.at[g + 1]], add=True)
            pltpu.async_copy(xws_hbm.at[sidx.at[g + 3]], rows1, sem1)

        pltpu.make_async_copy(xws_hbm.at[sidx.at[PB - 2]], rows0, sem0).wait()
        pltpu.sync_copy(rows0, accum_sh.at[didx.at[PB - 2]], add=True)
        pltpu.make_async_copy(xws_hbm.at[sidx.at[PB - 1]], rows1, sem1).wait()
        pltpu.sync_copy(rows1, accum_sh.at[didx.at[PB - 1]], add=True)

    plsc.subcore_barrier()
    pltpu.sync_copy(accum_sh.at[pl.ds(s * RPT, RPT)],
                    out_hbm.at[pl.ds(c * N + s * RPT, RPT)])

    @pl.when(s == NS - 1)
    def _():
        pltpu.sync_copy(accum_sh.at[pl.ds(NS * RPT, REM)],
                        out_hbm.at[pl.ds(c * N + NS * RPT, REM)])


_scatter_call = pl.kernel(
    _scatter_body,
    out_type=jax.ShapeDtypeStruct((NC * N, HH), jnp.float32),
    mesh=_mesh,
    scratch_types=[
        pltpu.VMEM((PB, EB), jnp.int32),
        pltpu.VMEM((PB, EB), jnp.int32),
        pltpu.VMEM((2, EB, HH), jnp.float32),
        pltpu.VMEM_SHARED((N, HH), jnp.float32),
        pltpu.SemaphoreType.DMA,
        pltpu.SemaphoreType.DMA,
    ],
)


def _dinv(degp_ref):
    return lax.rsqrt(degp_ref[0, :, 0:1] + degp_ref[1, :, 0:1] - 1.0)


def _mm1_body(x_ref, w_ref, degp_ref, out_ref):
    xw = jnp.dot(x_ref[...], w_ref[...], preferred_element_type=jnp.float32)
    d = _dinv(degp_ref)
    out_ref[0] = xw[:, :HH] * d
    out_ref[1] = xw[:, HH:] * d


def _mid_body(a_ref, degp_ref, b1_ref, w_ref, out_ref):
    d = _dinv(degp_ref)
    h0 = jnp.maximum(a_ref[0] * d + b1_ref[0, :HH], 0.0)
    h1 = jnp.maximum(a_ref[1] * d + b1_ref[0, HH:], 0.0)
    h = jnp.concatenate([h0, h1], axis=1)
    xw = jnp.dot(h, w_ref[...], preferred_element_type=jnp.float32)
    out_ref[0] = xw[:, :HH] * d
    out_ref[1] = xw[:, HH:] * d


def _fin_body(a_ref, degp_ref, b2_ref, wfc_ref, q_ref, bfc_ref, out_ref):
    d = _dinv(degp_ref)
    h0 = jnp.maximum(a_ref[0] * d + b2_ref[0, :HH], 0.0)
    h1 = jnp.maximum(a_ref[1] * d + b2_ref[0, HH:], 0.0)
    h = jnp.concatenate([h0, h1], axis=1)
    sc = jnp.dot(h, wfc_ref[:H, :], preferred_element_type=jnp.float32)
    const = jnp.dot(q_ref[...], wfc_ref[H:, :],
                    preferred_element_type=jnp.float32)[0, 0] + bfc_ref[0, 0]
    out_ref[...] = sc + const


def _row_specs():
    degp = pl.BlockSpec((NC, RB, 16), lambda i: (0, i, 0))
    half = pl.BlockSpec((NC, RB, HH), lambda i: (0, i, 0))
    return degp, half


def _mm1(x, W1, degp):
    dspec, half = _row_specs()
    return pl.pallas_call(
        _mm1_body,
        grid=(GRID,),
        in_specs=[
            pl.BlockSpec((RB, H), lambda i: (i, 0)),
            pl.BlockSpec((H, H), lambda i: (0, 0)),
            dspec,
        ],
        out_specs=half,
        out_shape=jax.ShapeDtypeStruct((NC, N, HH), jnp.float32),
    )(x, W1, degp)


def _mid(a, degp, b1r, W2):
    dspec, half = _row_specs()
    return pl.pallas_call(
        _mid_body,
        grid=(GRID,),
        in_specs=[
            half,
            dspec,
            pl.BlockSpec((1, H), lambda i: (0, 0)),
            pl.BlockSpec((H, H), lambda i: (0, 0)),
        ],
        out_specs=half,
        out_shape=jax.ShapeDtypeStruct((NC, N, HH), jnp.float32),
    )(a, degp, b1r, W2)


def _fin(a, degp, b2r, Wfc, qr, bfcr):
    dspec, half = _row_specs()
    return pl.pallas_call(
        _fin_body,
        grid=(GRID,),
        in_specs=[
            half,
            dspec,
            pl.BlockSpec((1, H), lambda i: (0, 0)),
            pl.BlockSpec((2 * H, 1), lambda i: (0, 0)),
            pl.BlockSpec((1, H), lambda i: (0, 0)),
            pl.BlockSpec((1, 1), lambda i: (0, 0)),
        ],
        out_specs=pl.BlockSpec((RB, 1), lambda i: (i, 0)),
        out_shape=jax.ShapeDtypeStruct((N, 1), jnp.float32),
    )(a, degp, b2r, Wfc, qr, bfcr)


@jax.jit
def kernel(x, edge_index, query_embedding, W1, b1, W2, b2, Wfc, bfc):
    src = edge_index[0].astype(jnp.int32)
    dst = edge_index[1].astype(jnp.int32)
    src4 = jnp.stack([src, src + N]).reshape(NC, NS, NB, EB)
    dst3 = dst.reshape(NS, NB, EB)
    dstd = dst.reshape(NC, NS, DEG_NB, EB)
    ones_hbm = jnp.ones((156, HH), jnp.float32)

    degp = _deg_call(dstd, ones_hbm)
    degp = degp[:, :, :16]
    xws1 = _mm1(x, W1, degp)
    a1 = _scatter_call(xws1.reshape(NC * N, HH), src4, dst3)
    xws2 = _mid(a1.reshape(NC, N, HH), degp, b1.reshape(1, H), W2)
    a2 = _scatter_call(xws2.reshape(NC * N, HH), src4, dst3)
    out = _fin(a2.reshape(NC, N, HH), degp, b2.reshape(1, H), Wfc,
               query_embedding.reshape(1, H), bfc.reshape(1, 1))
    return out.reshape(N)


# speedup vs baseline: 19.8575x; 1.1135x over previous
import functools

import jax
import jax.numpy as jnp
from jax import lax
from jax.experimental import pallas as pl
from jax.experimental.pallas import tpu as pltpu
from jax.experimental.pallas import tpu_sc as plsc

N = 10000
E = 160000
H = 256
HH = H // 2
NC, NS = 2, 16
EB = 125
NB = (E // NS) // EB
NP = 10240
EPT = E // (NC * NS)
NV = EPT // 16
TAIL = EPT - NV * 16
NPH = 2
PB = NB // NPH
RPT = 624
REM = N - NS * RPT

RB = 1000
GRID = N // RB

_mesh = plsc.VectorSubcoreMesh(core_axis_name="c", subcore_axis_name="s")


def _hist_body(dst_hbm, out_hbm, didx, hist):
    c = lax.axis_index("c")
    s = lax.axis_index("s")
    wid = c * NS + s

    @pl.loop(0, NP // 16)
    def _(i):
        hist[pl.ds(i * 16, 16)] = jnp.zeros((16,), jnp.float32)

    pltpu.sync_copy(dst_hbm.at[c, s], didx)
    ones = jnp.ones((16,), jnp.float32)
    full = lax.iota(jnp.int32, 16) >= 0

    @pl.loop(0, NV)
    def _(j):
        plsc.addupdate_scatter(hist, [didx[pl.ds(j * 16, 16)]], ones, mask=full)

    tl = didx[pl.ds(EPT - 16, 16)]
    mask = lax.iota(jnp.int32, 16) >= 16 - TAIL
    plsc.addupdate_scatter(hist, [tl], ones, mask=mask)
    pltpu.sync_copy(hist, out_hbm.at[wid])


_hist_call = pl.kernel(
    _hist_body,
    out_type=jax.ShapeDtypeStruct((NC * NS, NP), jnp.float32),
    mesh=_mesh,
    compiler_params=pltpu.CompilerParams(needs_layout_passes=False),
    scratch_types=[
        pltpu.VMEM((EPT,), jnp.int32),
        pltpu.VMEM((NP,), jnp.float32),
    ],
)


def _dcol_body(p_ref, out_ref):
    ones = jnp.ones((NC * NS, 1), jnp.float32)
    s = lax.dot_general(p_ref[...], ones, (((0,), (0,)), ((), ())),
                        preferred_element_type=jnp.float32)
    out_ref[...] = lax.rsqrt(1.0 + s[:N, :])


def _dcol(parts):
    return pl.pallas_call(
        _dcol_body,
        in_specs=[pl.BlockSpec((NC * NS, NP), lambda: (0, 0))],
        out_specs=pl.BlockSpec((N, 1), lambda: (0, 0)),
        out_shape=jax.ShapeDtypeStruct((N, 1), jnp.float32),
    )(parts)


def _scatter_body(xws_hbm, src_hbm, dst_hbm, out_hbm, sidx, didx, rows,
                  accum_sh, sem0, sem1):
    rows0, rows1 = rows.at[0], rows.at[1]
    c = lax.axis_index("c")
    s = lax.axis_index("s")
    pltpu.sync_copy(xws_hbm.at[pl.ds(c * N + s * RPT, RPT)],
                    accum_sh.at[pl.ds(s * RPT, RPT)])

    @pl.when(s == NS - 1)
    def _():
        pltpu.sync_copy(xws_hbm.at[pl.ds(c * N + NS * RPT, REM)],
                        accum_sh.at[pl.ds(NS * RPT, REM)])

    plsc.subcore_barrier()

    for p in range(NPH):
        pltpu.sync_copy(src_hbm.at[c, s, pl.ds(p * PB, PB)], sidx)
        pltpu.sync_copy(dst_hbm.at[s, pl.ds(p * PB, PB)], didx)
        pltpu.async_copy(xws_hbm.at[sidx.at[0]], rows0, sem0)
        pltpu.async_copy(xws_hbm.at[sidx.at[1]], rows1, sem1)

        @pl.loop(0, PB - 2, step=2)
        def _(g):
            pltpu.make_async_copy(xws_hbm.at[sidx.at[g]], rows0, sem0).wait()
            pltpu.sync_copy(rows0, accum_sh.at[didx.at[g]], add=True)
            pltpu.async_copy(xws_hbm.at[sidx.at[g + 2]], rows0, sem0)
            pltpu.make_async_copy(xws_hbm.at[sidx.at[g + 1]], rows1, sem1).wait()
            pltpu.sync_copy(rows1, accum_sh.at[didx.at[g + 1]], add=True)
            pltpu.async_copy(xws_hbm.at[sidx.at[g + 3]], rows1, sem1)

        pltpu.make_async_copy(xws_hbm.at[sidx.at[PB - 2]], rows0, sem0).wait()
        pltpu.sync_copy(rows0, accum_sh.at[didx.at[PB - 2]], add=True)
        pltpu.make_async_copy(xws_hbm.at[sidx.at[PB - 1]], rows1, sem1).wait()
        pltpu.sync_copy(rows1, accum_sh.at[didx.at[PB - 1]], add=True)

    plsc.subcore_barrier()
    pltpu.sync_copy(accum_sh.at[pl.ds(s * RPT, RPT)],
                    out_hbm.at[pl.ds(c * N + s * RPT, RPT)])

    @pl.when(s == NS - 1)
    def _():
        pltpu.sync_copy(accum_sh.at[pl.ds(NS * RPT, REM)],
                        out_hbm.at[pl.ds(c * N + NS * RPT, REM)])


_scatter_call = pl.kernel(
    _scatter_body,
    out_type=jax.ShapeDtypeStruct((NC * N, HH), jnp.float32),
    mesh=_mesh,
    scratch_types=[
        pltpu.VMEM((PB, EB), jnp.int32),
        pltpu.VMEM((PB, EB), jnp.int32),
        pltpu.VMEM((2, EB, HH), jnp.float32),
        pltpu.VMEM_SHARED((N, HH), jnp.float32),
        pltpu.SemaphoreType.DMA,
        pltpu.SemaphoreType.DMA,
    ],
)


def _mm1_body(x_ref, w_ref, d_ref, out_ref):
    xw = jnp.dot(x_ref[...], w_ref[...], preferred_element_type=jnp.float32)
    d = d_ref[...]
    out_ref[0] = xw[:, :HH] * d
    out_ref[1] = xw[:, HH:] * d


def _mid_body(a_ref, d_ref, b1_ref, w_ref, out_ref):
    d = d_ref[...]
    h0 = jnp.maximum(a_ref[0] * d + b1_ref[0, :HH], 0.0)
    h1 = jnp.maximum(a_ref[1] * d + b1_ref[0, HH:], 0.0)
    h = jnp.concatenate([h0, h1], axis=1)
    xw = jnp.dot(h, w_ref[...], preferred_element_type=jnp.float32)
    out_ref[0] = xw[:, :HH] * d
    out_ref[1] = xw[:, HH:] * d


def _fin_body(a_ref, d_ref, b2_ref, wfc_ref, q_ref, bfc_ref, out_ref):
    d = d_ref[...]
    h0 = jnp.maximum(a_ref[0] * d + b2_ref[0, :HH], 0.0)
    h1 = jnp.maximum(a_ref[1] * d + b2_ref[0, HH:], 0.0)
    h = jnp.concatenate([h0, h1], axis=1)
    sc = jnp.dot(h, wfc_ref[:H, :], preferred_element_type=jnp.float32)
    const = jnp.dot(q_ref[...], wfc_ref[H:, :],
                    preferred_element_type=jnp.float32)[0, 0] + bfc_ref[0, 0]
    out_ref[...] = sc + const


def _row_specs():
    dcol = pl.BlockSpec((RB, 1), lambda i: (i, 0))
    half = pl.BlockSpec((NC, RB, HH), lambda i: (0, i, 0))
    return dcol, half


def _mm1(x, W1, dc):
    dspec, half = _row_specs()
    return pl.pallas_call(
        _mm1_body,
        grid=(GRID,),
        in_specs=[
            pl.BlockSpec((RB, H), lambda i: (i, 0)),
            pl.BlockSpec((H, H), lambda i: (0, 0)),
            dspec,
        ],
        out_specs=half,
        out_shape=jax.ShapeDtypeStruct((NC, N, HH), jnp.float32),
    )(x, W1, dc)


def _mid(a, dc, b1r, W2):
    dspec, half = _row_specs()
    return pl.pallas_call(
        _mid_body,
        grid=(GRID,),
        in_specs=[
            half,
            dspec,
            pl.BlockSpec((1, H), lambda i: (0, 0)),
            pl.BlockSpec((H, H), lambda i: (0, 0)),
        ],
        out_specs=half,
        out_shape=jax.ShapeDtypeStruct((NC, N, HH), jnp.float32),
    )(a, dc, b1r, W2)


def _fin(a, dc, b2r, Wfc, qr, bfcr):
    dspec, half = _row_specs()
    return pl.pallas_call(
        _fin_body,
        grid=(GRID,),
        in_specs=[
            half,
            dspec,
            pl.BlockSpec((1, H), lambda i: (0, 0)),
            pl.BlockSpec((2 * H, 1), lambda i: (0, 0)),
            pl.BlockSpec((1, H), lambda i: (0, 0)),
            pl.BlockSpec((1, 1), lambda i: (0, 0)),
        ],
        out_specs=pl.BlockSpec((RB, 1), lambda i: (i, 0)),
        out_shape=jax.ShapeDtypeStruct((N, 1), jnp.float32),
    )(a, dc, b2r, Wfc, qr, bfcr)


@jax.jit
def kernel(x, edge_index, query_embedding, W1, b1, W2, b2, Wfc, bfc):
    src = edge_index[0].astype(jnp.int32)
    dst = edge_index[1].astype(jnp.int32)
    src4 = jnp.stack([src, src + N]).reshape(NC, NS, NB, EB)
    dst3 = dst.reshape(NS, NB, EB)
    dstd = dst.reshape(NC, NS, EPT)

    parts = _hist_call(dstd)
    dc = _dcol(parts)
    xws1 = _mm1(x, W1, dc)
    a1 = _scatter_call(xws1.reshape(NC * N, HH), src4, dst3)
    xws2 = _mid(a1.reshape(NC, N, HH), dc, b1.reshape(1, H), W2)
    a2 = _scatter_call(xws2.reshape(NC * N, HH), src4, dst3)
    out = _fin(a2.reshape(NC, N, HH), dc, b2.reshape(1, H), Wfc,
               query_embedding.reshape(1, H), bfc.reshape(1, 1))
    return out.reshape(N)


# speedup vs baseline: 20.1448x; 1.0145x over previous
import functools

import jax
import jax.numpy as jnp
from jax import lax
from jax.experimental import pallas as pl
from jax.experimental.pallas import tpu as pltpu
from jax.experimental.pallas import tpu_sc as plsc

N = 10000
E = 160000
H = 256
HH = H // 2
NC, NS = 2, 16
EB = 125
NB = (E // NS) // EB
NP = 10240
EPT = E // (NC * NS)
NV = EPT // 16
TAIL = EPT - NV * 16
NPH = 2
PB = NB // NPH
RPT = 624
REM = N - NS * RPT

RB = 1000
GRID = N // RB

_mesh = plsc.VectorSubcoreMesh(core_axis_name="c", subcore_axis_name="s")


def _hist_body(ei_hbm, out_hbm, didx, hist):
    c = lax.axis_index("c")
    s = lax.axis_index("s")
    wid = c * NS + s

    @pl.loop(0, NP // 16)
    def _(i):
        hist[pl.ds(i * 16, 16)] = jnp.zeros((16,), jnp.float32)

    pltpu.sync_copy(ei_hbm.at[1, c, s], didx)
    ones = jnp.ones((16,), jnp.float32)
    full = lax.iota(jnp.int32, 16) >= 0

    @pl.loop(0, NV)
    def _(j):
        plsc.addupdate_scatter(hist, [didx[pl.ds(j * 16, 16)]], ones, mask=full)

    tl = didx[pl.ds(EPT - 16, 16)]
    mask = lax.iota(jnp.int32, 16) >= 16 - TAIL
    plsc.addupdate_scatter(hist, [tl], ones, mask=mask)
    pltpu.sync_copy(hist, out_hbm.at[wid])


_hist_call = pl.kernel(
    _hist_body,
    out_type=jax.ShapeDtypeStruct((NC * NS, NP), jnp.float32),
    mesh=_mesh,
    compiler_params=pltpu.CompilerParams(needs_layout_passes=False),
    scratch_types=[
        pltpu.VMEM((EPT,), jnp.int32),
        pltpu.VMEM((NP,), jnp.float32),
    ],
)


def _dcol_body(p_ref, out_ref):
    ones = jnp.ones((NC * NS, 1), jnp.float32)
    s = lax.dot_general(p_ref[...], ones, (((0,), (0,)), ((), ())),
                        preferred_element_type=jnp.float32)
    out_ref[...] = lax.rsqrt(1.0 + s[:N, :])


def _dcol(parts):
    return pl.pallas_call(
        _dcol_body,
        in_specs=[pl.BlockSpec((NC * NS, NP), lambda: (0, 0))],
        out_specs=pl.BlockSpec((N, 1), lambda: (0, 0)),
        out_shape=jax.ShapeDtypeStruct((N, 1), jnp.float32),
    )(parts)


def _scatter_body(xws_hbm, ei_hbm, out_hbm, sidx, didx, rows,
                  accum_sh, sem0, sem1):
    rows0, rows1 = rows.at[0], rows.at[1]
    c = lax.axis_index("c")
    s = lax.axis_index("s")
    tbl = xws_hbm.at[c]
    pltpu.sync_copy(tbl.at[pl.ds(s * RPT, RPT)],
                    accum_sh.at[pl.ds(s * RPT, RPT)])

    @pl.when(s == NS - 1)
    def _():
        pltpu.sync_copy(tbl.at[pl.ds(NS * RPT, REM)],
                        accum_sh.at[pl.ds(NS * RPT, REM)])

    plsc.subcore_barrier()

    for p in range(NPH):
        pltpu.sync_copy(ei_hbm.at[0, s, pl.ds(p * PB, PB)], sidx)
        pltpu.sync_copy(ei_hbm.at[1, s, pl.ds(p * PB, PB)], didx)
        pltpu.async_copy(tbl.at[sidx.at[0]], rows0, sem0)
        pltpu.async_copy(tbl.at[sidx.at[1]], rows1, sem1)

        @pl.loop(0, PB - 2, step=2)
        def _(g):
            pltpu.make_async_copy(tbl.at[sidx.at[g]], rows0, sem0).wait()
            pltpu.sync_copy(rows0, accum_sh.at[didx.at[g]], add=True)
            pltpu.async_copy(tbl.at[sidx.at[g + 2]], rows0, sem0)
            pltpu.make_async_copy(tbl.at[sidx.at[g + 1]], rows1, sem1).wait()
            pltpu.sync_copy(rows1, accum_sh.at[didx.at[g + 1]], add=True)
            pltpu.async_copy(tbl.at[sidx.at[g + 3]], rows1, sem1)

        pltpu.make_async_copy(tbl.at[sidx.at[PB - 2]], rows0, sem0).wait()
        pltpu.sync_copy(rows0, accum_sh.at[didx.at[PB - 2]], add=True)
        pltpu.make_async_copy(tbl.at[sidx.at[PB - 1]], rows1, sem1).wait()
        pltpu.sync_copy(rows1, accum_sh.at[didx.at[PB - 1]], add=True)

    plsc.subcore_barrier()
    pltpu.sync_copy(accum_sh.at[pl.ds(s * RPT, RPT)],
                    out_hbm.at[c, pl.ds(s * RPT, RPT)])

    @pl.when(s == NS - 1)
    def _():
        pltpu.sync_copy(accum_sh.at[pl.ds(NS * RPT, REM)],
                        out_hbm.at[c, pl.ds(NS * RPT, REM)])


_scatter_call = pl.kernel(
    _scatter_body,
    out_type=jax.ShapeDtypeStruct((NC, N, HH), jnp.float32),
    mesh=_mesh,
    scratch_types=[
        pltpu.VMEM((PB, EB), jnp.int32),
        pltpu.VMEM((PB, EB), jnp.int32),
        pltpu.VMEM((2, EB, HH), jnp.float32),
        pltpu.VMEM_SHARED((N, HH), jnp.float32),
        pltpu.SemaphoreType.DMA,
        pltpu.SemaphoreType.DMA,
    ],
)


def _mm1_body(x_ref, w_ref, d_ref, out_ref):
    xw = jnp.dot(x_ref[...], w_ref[...], preferred_element_type=jnp.float32)
    d = d_ref[...]
    out_ref[0] = xw[:, :HH] * d
    out_ref[1] = xw[:, HH:] * d


def _mid_body(a_ref, d_ref, b1_ref, w_ref, out_ref):
    d = d_ref[...]
    h0 = jnp.maximum(a_ref[0] * d + b1_ref[0, :HH], 0.0)
    h1 = jnp.maximum(a_ref[1] * d + b1_ref[0, HH:], 0.0)
    h = jnp.concatenate([h0, h1], axis=1)
    xw = jnp.dot(h, w_ref[...], preferred_element_type=jnp.float32)
    out_ref[0] = xw[:, :HH] * d
    out_ref[1] = xw[:, HH:] * d


def _fin_body(a_ref, d_ref, b2_ref, wfc_ref, q_ref, bfc_ref, out_ref):
    d = d_ref[...]
    h0 = jnp.maximum(a_ref[0] * d + b2_ref[0, :HH], 0.0)
    h1 = jnp.maximum(a_ref[1] * d + b2_ref[0, HH:], 0.0)
    h = jnp.concatenate([h0, h1], axis=1)
    sc = jnp.dot(h, wfc_ref[:H, :], preferred_element_type=jnp.float32)
    const = jnp.dot(q_ref[...], wfc_ref[H:, :],
                    preferred_element_type=jnp.float32)[0, 0] + bfc_ref[0, 0]
    out_ref[...] = sc + const


def _row_specs():
    dcol = pl.BlockSpec((RB, 1), lambda i: (i, 0))
    half = pl.BlockSpec((NC, RB, HH), lambda i: (0, i, 0))
    return dcol, half


def _mm1(x, W1, dc):
    dspec, half = _row_specs()
    return pl.pallas_call(
        _mm1_body,
        grid=(GRID,),
        in_specs=[
            pl.BlockSpec((RB, H), lambda i: (i, 0)),
            pl.BlockSpec((H, H), lambda i: (0, 0)),
            dspec,
        ],
        out_specs=half,
        out_shape=jax.ShapeDtypeStruct((NC, N, HH), jnp.float32),
    )(x, W1, dc)


def _mid(a, dc, b1r, W2):
    dspec, half = _row_specs()
    return pl.pallas_call(
        _mid_body,
        grid=(GRID,),
        in_specs=[
            half,
            dspec,
            pl.BlockSpec((1, H), lambda i: (0, 0)),
            pl.BlockSpec((H, H), lambda i: (0, 0)),
        ],
        out_specs=half,
        out_shape=jax.ShapeDtypeStruct((NC, N, HH), jnp.float32),
    )(a, dc, b1r, W2)


def _fin(a, dc, b2r, Wfc, qr, bfcr):
    dspec, half = _row_specs()
    return pl.pallas_call(
        _fin_body,
        grid=(GRID,),
        in_specs=[
            half,
            dspec,
            pl.BlockSpec((1, H), lambda i: (0, 0)),
            pl.BlockSpec((2 * H, 1), lambda i: (0, 0)),
            pl.BlockSpec((1, H), lambda i: (0, 0)),
            pl.BlockSpec((1, 1), lambda i: (0, 0)),
        ],
        out_specs=pl.BlockSpec((RB, 1), lambda i: (i, 0)),
        out_shape=jax.ShapeDtypeStruct((N, 1), jnp.float32),
    )(a, dc, b2r, Wfc, qr, bfcr)


@jax.jit
def kernel(x, edge_index, query_embedding, W1, b1, W2, b2, Wfc, bfc):
    ei4 = edge_index.reshape(2, NS, NB, EB)
    eih = edge_index.reshape(2, NC, NS, EPT)

    parts = _hist_call(eih)
    dc = _dcol(parts)
    xws1 = _mm1(x, W1, dc)
    a1 = _scatter_call(xws1, ei4)
    xws2 = _mid(a1, dc, b1.reshape(1, H), W2)
    a2 = _scatter_call(xws2, ei4)
    out = _fin(a2, dc, b2.reshape(1, H), Wfc,
               query_embedding.reshape(1, H), bfc.reshape(1, 1))
    return out.reshape(N)


# speedup vs baseline: 20.5023x; 1.0177x over previous
import functools

import jax
import jax.numpy as jnp
from jax import lax
from jax.experimental import pallas as pl
from jax.experimental.pallas import tpu as pltpu
from jax.experimental.pallas import tpu_sc as plsc

N = 10000
E = 160000
H = 256
HH = H // 2
NC, NS = 2, 16
EB = 125
NB = (E // NS) // EB
NP = 10240
EPT = E // (NC * NS)
NV = EPT // 16
TAIL = EPT - NV * 16
NPH = 2
PB = NB // NPH
RPT = 624
REM = N - NS * RPT

RB = 1000
GRID = N // RB

_mesh = plsc.VectorSubcoreMesh(core_axis_name="c", subcore_axis_name="s")


def _hist_body(ei_hbm, out_hbm, didx, hist):
    c = lax.axis_index("c")
    s = lax.axis_index("s")
    wid = c * NS + s

    @pl.loop(0, NP // 16)
    def _(i):
        hist[pl.ds(i * 16, 16)] = jnp.zeros((16,), jnp.float32)

    pltpu.sync_copy(ei_hbm.at[1, c, s], didx)
    ones = jnp.ones((16,), jnp.float32)
    full = lax.iota(jnp.int32, 16) >= 0

    @pl.loop(0, NV)
    def _(j):
        plsc.addupdate_scatter(hist, [didx[pl.ds(j * 16, 16)]], ones, mask=full)

    tl = didx[pl.ds(EPT - 16, 16)]
    mask = lax.iota(jnp.int32, 16) >= 16 - TAIL
    plsc.addupdate_scatter(hist, [tl], ones, mask=mask)
    pltpu.sync_copy(hist, out_hbm.at[wid])


_hist_call = pl.kernel(
    _hist_body,
    out_type=jax.ShapeDtypeStruct((NC * NS, NP), jnp.float32),
    mesh=_mesh,
    compiler_params=pltpu.CompilerParams(needs_layout_passes=False),
    scratch_types=[
        pltpu.VMEM((EPT,), jnp.int32),
        pltpu.VMEM((NP,), jnp.float32),
    ],
)


def _dcol_body(p_ref, out_ref):
    ones = jnp.ones((NC * NS, 1), jnp.float32)
    s = lax.dot_general(p_ref[...], ones, (((0,), (0,)), ((), ())),
                        preferred_element_type=jnp.float32)
    out_ref[...] = lax.rsqrt(1.0 + s[:N, :])


def _dcol(parts):
    return pl.pallas_call(
        _dcol_body,
        in_specs=[pl.BlockSpec((NC * NS, NP), lambda: (0, 0))],
        out_specs=pl.BlockSpec((N, 1), lambda: (0, 0)),
        out_shape=jax.ShapeDtypeStruct((N, 1), jnp.float32),
    )(parts)


def _scatter_body(xws_hbm, ei_hbm, out_hbm, sidx, didx, rows,
                  accum_sh, sem0, sem1):
    rows0, rows1 = rows.at[0], rows.at[1]
    c = lax.axis_index("c")
    s = lax.axis_index("s")
    tbl = xws_hbm.at[c]
    pltpu.async_copy(tbl.at[pl.ds(s * RPT, RPT)],
                     accum_sh.at[pl.ds(s * RPT, RPT)], sem0)

    @pl.when(s == NS - 1)
    def _():
        pltpu.sync_copy(tbl.at[pl.ds(NS * RPT, REM)],
                        accum_sh.at[pl.ds(NS * RPT, REM)])

    pltpu.sync_copy(ei_hbm.at[0, s, pl.ds(0, PB)], sidx)
    pltpu.sync_copy(ei_hbm.at[1, s, pl.ds(0, PB)], didx)
    pltpu.make_async_copy(tbl.at[pl.ds(s * RPT, RPT)],
                          accum_sh.at[pl.ds(s * RPT, RPT)], sem0).wait()
    pltpu.async_copy(tbl.at[sidx.at[0]], rows0, sem0)
    pltpu.async_copy(tbl.at[sidx.at[1]], rows1, sem1)
    plsc.subcore_barrier()

    for p in range(NPH):
        if p > 0:
            pltpu.sync_copy(ei_hbm.at[0, s, pl.ds(p * PB, PB)], sidx)
            pltpu.sync_copy(ei_hbm.at[1, s, pl.ds(p * PB, PB)], didx)
            pltpu.async_copy(tbl.at[sidx.at[0]], rows0, sem0)
            pltpu.async_copy(tbl.at[sidx.at[1]], rows1, sem1)

        @pl.loop(0, PB - 2, step=2)
        def _(g):
            pltpu.make_async_copy(tbl.at[sidx.at[g]], rows0, sem0).wait()
            pltpu.sync_copy(rows0, accum_sh.at[didx.at[g]], add=True)
            pltpu.async_copy(tbl.at[sidx.at[g + 2]], rows0, sem0)
            pltpu.make_async_copy(tbl.at[sidx.at[g + 1]], rows1, sem1).wait()
            pltpu.sync_copy(rows1, accum_sh.at[didx.at[g + 1]], add=True)
            pltpu.async_copy(tbl.at[sidx.at[g + 3]], rows1, sem1)

        pltpu.make_async_copy(tbl.at[sidx.at[PB - 2]], rows0, sem0).wait()
        pltpu.sync_copy(rows0, accum_sh.at[didx.at[PB - 2]], add=True)
        pltpu.make_async_copy(tbl.at[sidx.at[PB - 1]], rows1, sem1).wait()
        pltpu.sync_copy(rows1, accum_sh.at[didx.at[PB - 1]], add=True)

    plsc.subcore_barrier()
    pltpu.sync_copy(accum_sh.at[pl.ds(s * RPT, RPT)],
                    out_hbm.at[c, pl.ds(s * RPT, RPT)])

    @pl.when(s == NS - 1)
    def _():
        pltpu.sync_copy(accum_sh.at[pl.ds(NS * RPT, REM)],
                        out_hbm.at[c, pl.ds(NS * RPT, REM)])


_scatter_call = pl.kernel(
    _scatter_body,
    out_type=jax.ShapeDtypeStruct((NC, N, HH), jnp.float32),
    mesh=_mesh,
    scratch_types=[
        pltpu.VMEM((PB, EB), jnp.int32),
        pltpu.VMEM((PB, EB), jnp.int32),
        pltpu.VMEM((2, EB, HH), jnp.float32),
        pltpu.VMEM_SHARED((N, HH), jnp.float32),
        pltpu.SemaphoreType.DMA,
        pltpu.SemaphoreType.DMA,
    ],
)


def _mm1_body(x_ref, w_ref, d_ref, out_ref):
    xw = jnp.dot(x_ref[...], w_ref[...], preferred_element_type=jnp.float32)
    d = d_ref[...]
    out_ref[0] = xw[:, :HH] * d
    out_ref[1] = xw[:, HH:] * d


def _mid_body(a_ref, d_ref, b1_ref, w_ref, out_ref):
    d = d_ref[...]
    h0 = jnp.maximum(a_ref[0] * d + b1_ref[0, :HH], 0.0)
    h1 = jnp.maximum(a_ref[1] * d + b1_ref[0, HH:], 0.0)
    h = jnp.concatenate([h0, h1], axis=1)
    xw = jnp.dot(h, w_ref[...], preferred_element_type=jnp.float32)
    out_ref[0] = xw[:, :HH] * d
    out_ref[1] = xw[:, HH:] * d


def _fin_body(a_ref, d_ref, b2_ref, wfc_ref, q_ref, bfc_ref, out_ref):
    d = d_ref[...]
    h0 = jnp.maximum(a_ref[0] * d + b2_ref[0, :HH], 0.0)
    h1 = jnp.maximum(a_ref[1] * d + b2_ref[0, HH:], 0.0)
    h = jnp.concatenate([h0, h1], axis=1)
    sc = jnp.dot(h, wfc_ref[:H, :], preferred_element_type=jnp.float32)
    const = jnp.dot(q_ref[...], wfc_ref[H:, :],
                    preferred_element_type=jnp.float32)[0, 0] + bfc_ref[0, 0]
    out_ref[...] = sc + const


def _row_specs():
    dcol = pl.BlockSpec((RB, 1), lambda i: (i, 0))
    half = pl.BlockSpec((NC, RB, HH), lambda i: (0, i, 0))
    return dcol, half


def _mm1(x, W1, dc):
    dspec, half = _row_specs()
    return pl.pallas_call(
        _mm1_body,
        grid=(GRID,),
        in_specs=[
            pl.BlockSpec((RB, H), lambda i: (i, 0)),
            pl.BlockSpec((H, H), lambda i: (0, 0)),
            dspec,
        ],
        out_specs=half,
        out_shape=jax.ShapeDtypeStruct((NC, N, HH), jnp.float32),
    )(x, W1, dc)


def _mid(a, dc, b1r, W2):
    dspec, half = _row_specs()
    return pl.pallas_call(
        _mid_body,
        grid=(GRID,),
        in_specs=[
            half,
            dspec,
            pl.BlockSpec((1, H), lambda i: (0, 0)),
            pl.BlockSpec((H, H), lambda i: (0, 0)),
        ],
        out_specs=half,
        out_shape=jax.ShapeDtypeStruct((NC, N, HH), jnp.float32),
    )(a, dc, b1r, W2)


def _fin(a, dc, b2r, Wfc, qr, bfcr):
    dspec, half = _row_specs()
    return pl.pallas_call(
        _fin_body,
        grid=(GRID,),
        in_specs=[
            half,
            dspec,
            pl.BlockSpec((1, H), lambda i: (0, 0)),
            pl.BlockSpec((2 * H, 1), lambda i: (0, 0)),
            pl.BlockSpec((1, H), lambda i: (0, 0)),
            pl.BlockSpec((1, 1), lambda i: (0, 0)),
        ],
        out_specs=pl.BlockSpec((RB, 1), lambda i: (i, 0)),
        out_shape=jax.ShapeDtypeStruct((N, 1), jnp.float32),
    )(a, dc, b2r, Wfc, qr, bfcr)


@jax.jit
def kernel(x, edge_index, query_embedding, W1, b1, W2, b2, Wfc, bfc):
    ei4 = edge_index.reshape(2, NS, NB, EB)
    eih = edge_index.reshape(2, NC, NS, EPT)

    parts = _hist_call(eih)
    dc = _dcol(parts)
    xws1 = _mm1(x, W1, dc)
    a1 = _scatter_call(xws1, ei4)
    xws2 = _mid(a1, dc, b1.reshape(1, H), W2)
    a2 = _scatter_call(xws2, ei4)
    out = _fin(a2, dc, b2.reshape(1, H), Wfc,
               query_embedding.reshape(1, H), bfc.reshape(1, 1))
    return out.reshape(N)
